# speedup vs baseline: 3.3412x; 3.3412x over previous
import functools

import jax
import jax.numpy as jnp
from jax import lax
from jax.experimental import pallas as pl
from jax.experimental.pallas import tpu as pltpu
from jax.experimental.pallas import tpu_sc as plsc

B = 16384
F = 26
E = 128
R = B * F
NW = 32
RW = R // NW
C = 128
NCH = RW // C
NBUF = 4

_mesh = plsc.VectorSubcoreMesh(core_axis_name="c", subcore_axis_name="s")


@functools.partial(
    pl.kernel,
    out_type=jax.ShapeDtypeStruct((NW * NCH, C, E), jnp.float32),
    mesh=_mesh,
    scratch_types=[
        pltpu.VMEM((NCH, C), jnp.int32),
        pltpu.VMEM((NBUF, C, E), jnp.float32),
    ]
    + [pltpu.SemaphoreType.DMA] * (2 * NBUF),
)
def _embed(x_hbm, w_hbm, out_hbm, idx_v, rows_v, *sems):
    gsem = sems[:NBUF]
    osem = sems[NBUF:]
    wid = lax.axis_index("s") * 2 + lax.axis_index("c")

    pltpu.sync_copy(x_hbm.at[wid], idx_v)

    lanes = lax.iota(jnp.int32, 16)
    base = wid * RW

    @pl.loop(0, NCH)
    def _offsets(j):
        row0 = base + j * C
        for k in range(C // 16):
            sl = pl.ds(k * 16, 16)
            p = row0 + k * 16 + lanes
            idx_v[j, sl] = idx_v[j, sl] + lax.rem(p, 26) * 1000

    def start_gather(g, b):
        pltpu.async_copy(w_hbm.at[idx_v.at[g]], rows_v.at[b], gsem[b])

    def wait_gather(b):
        pltpu.make_async_copy(w_hbm.at[idx_v.at[0]], rows_v.at[b], gsem[b]).wait()

    def start_out(g, b):
        pltpu.async_copy(rows_v.at[b], out_hbm.at[wid * NCH + g], osem[b])

    def wait_out(b):
        pltpu.make_async_copy(rows_v.at[b], out_hbm.at[0], osem[b]).wait()

    for b in range(NBUF):
        start_gather(b, b)

    @pl.loop(0, NCH, step=NBUF)
    def _chunks(g0):
        for b in range(NBUF):
            g = g0 + b
            wait_gather(b)
            start_out(g, b)

            @pl.when(g0 + NBUF < NCH)
            def _():
                wait_out(b)
                start_gather(g + NBUF, b)

    for b in range(NBUF):
        wait_out(b)


def kernel(x, weight):
    out = _embed(x.reshape(NW, NCH, C), weight)
    return out.reshape(B, F, E)


# speedup vs baseline: 4.9710x; 1.4878x over previous
import functools

import jax
import jax.numpy as jnp
from jax import lax
from jax.experimental import pallas as pl
from jax.experimental.pallas import tpu as pltpu
from jax.experimental.pallas import tpu_sc as plsc

B = 16384
F = 26
FP = 32
E = 128
R = B * F
NW = 32
RW = R // NW
BW = B // NW
CB = 4
C = CB * F
CG = C + (FP - F)
NCH = BW // CB
NBUF = 8

_mesh = plsc.VectorSubcoreMesh(core_axis_name="c", subcore_axis_name="s")


@functools.partial(
    pl.kernel,
    out_type=jax.ShapeDtypeStruct((B * FP, E), jnp.float32),
    mesh=_mesh,
    scratch_types=[
        pltpu.VMEM((RW + 16,), jnp.int32),
        pltpu.VMEM((NBUF, CG, E), jnp.float32),
    ]
    + [pltpu.SemaphoreType.DMA] * (2 * NBUF),
)
def _embed(x_hbm, w_hbm, out_hbm, idx_v, rows_v, *sems):
    gsem = sems[:NBUF]
    osem = sems[NBUF:]
    wid = lax.axis_index("s") * 2 + lax.axis_index("c")

    pltpu.sync_copy(x_hbm.at[wid], idx_v.at[pl.ds(0, RW)])

    lanes = lax.iota(jnp.int32, 16)
    base = wid * RW

    @pl.loop(0, RW // 16, unroll=8)
    def _offsets(i):
        sl = pl.ds(i * 16, 16)
        p = base + i * 16 + lanes
        idx_v[sl] = idx_v[sl] + lax.rem(p, 26) * 1000

    idx_v[pl.ds(RW, 16)] = jnp.zeros((16,), jnp.int32)

    def start_gather(g, b):
        pltpu.async_copy(
            w_hbm.at[idx_v.at[pl.ds(g * C, CG)]], rows_v.at[b], gsem[b]
        )

    def wait_gather(b):
        pltpu.make_async_copy(
            w_hbm.at[idx_v.at[pl.ds(0, CG)]], rows_v.at[b], gsem[b]
        ).wait()

    def start_out(g, b):
        b0 = wid * BW + g * CB
        for i in range(CB):
            pltpu.async_copy(
                rows_v.at[b, pl.ds(i * F, FP)],
                out_hbm.at[pl.ds((b0 + i) * FP, FP)],
                osem[b],
            )

    def wait_out(b):
        for _ in range(CB):
            pltpu.make_async_copy(
                rows_v.at[b, pl.ds(0, FP)],
                out_hbm.at[pl.ds(0, FP)],
                osem[b],
            ).wait()

    for b in range(NBUF):
        start_gather(b, b)

    @pl.loop(0, NCH, step=NBUF)
    def _chunks(g0):
        for b in range(NBUF):
            g = g0 + b
            wait_gather(b)
            start_out(g, b)

            @pl.when(g0 + NBUF < NCH)
            def _():
                wait_out(b)
                start_gather(g + NBUF, b)

    for b in range(NBUF):
        wait_out(b)


def kernel(x, weight):
    out = _embed(x.reshape(NW, RW), weight)
    return out.reshape(B, FP, E)[:, :F, :]


# speedup vs baseline: 5.6613x; 1.1389x over previous
import functools

import jax
import jax.numpy as jnp
from jax import lax
from jax.experimental import pallas as pl
from jax.experimental.pallas import tpu as pltpu
from jax.experimental.pallas import tpu_sc as plsc

B = 16384
F = 26
E = 128
R = B * F
NW = 32
RW = R // NW
BW = B // NW
CB = 4
C = CB * F
NCH = BW // CB
NBUF = 8

_mesh = plsc.VectorSubcoreMesh(core_axis_name="c", subcore_axis_name="s")


@functools.partial(
    pl.kernel,
    out_type=jax.ShapeDtypeStruct((B, F, E), jnp.float32),
    mesh=_mesh,
    scratch_types=[
        pltpu.VMEM((RW,), jnp.int32),
        pltpu.VMEM((NBUF, C, E), jnp.float32),
    ]
    + [pltpu.SemaphoreType.DMA] * (2 * NBUF),
    compiler_params=pltpu.CompilerParams(use_tc_tiling_on_sc=True),
)
def _embed(x_hbm, w_hbm, out_hbm, idx_v, rows_v, *sems):
    gsem = sems[:NBUF]
    osem = sems[NBUF:]
    wid = lax.axis_index("s") * 2 + lax.axis_index("c")

    pltpu.sync_copy(x_hbm.at[wid], idx_v)

    lanes = lax.iota(jnp.int32, 16)
    base = wid * RW

    @pl.loop(0, RW // 16, unroll=8)
    def _offsets(i):
        sl = pl.ds(i * 16, 16)
        p = base + i * 16 + lanes
        idx_v[sl] = idx_v[sl] + lax.rem(p, 26) * 1000

    def start_gather(g, b):
        pltpu.async_copy(
            w_hbm.at[idx_v.at[pl.ds(g * C, C)]], rows_v.at[b], gsem[b]
        )

    def wait_gather(b):
        pltpu.make_async_copy(
            w_hbm.at[idx_v.at[pl.ds(0, C)]], rows_v.at[b], gsem[b]
        ).wait()

    def start_out(g, b):
        b0 = wid * BW + g * CB
        for i in range(CB):
            pltpu.async_copy(
                rows_v.at[b, pl.ds(i * F, F)],
                out_hbm.at[b0 + i],
                osem[b],
            )

    def wait_out(b):
        for _ in range(CB):
            pltpu.make_async_copy(
                rows_v.at[b, pl.ds(0, F)],
                out_hbm.at[0],
                osem[b],
            ).wait()

    for b in range(NBUF):
        start_gather(b, b)

    @pl.loop(0, NCH, step=NBUF)
    def _chunks(g0):
        for b in range(NBUF):
            g = g0 + b
            wait_gather(b)
            start_out(g, b)

            @pl.when(g0 + NBUF < NCH)
            def _():
                wait_out(b)
                start_gather(g + NBUF, b)

    for b in range(NBUF):
        wait_out(b)


def kernel(x, weight):
    return _embed(x.reshape(NW, RW), weight)
